# Initial kernel scaffold; baseline (speedup 1.0000x reference)
import jax
import jax.numpy as jnp
from jax.experimental import pallas as pl


def kernel(x, router_logits, per_expert_scale, w_gate, w_up, w_down):
    raise NotImplementedError("write your pallas kernel here")


# speedup vs baseline: 2.2133x; 2.2133x over previous
import functools

import jax
import jax.numpy as jnp
from jax import lax
from jax.experimental import pallas as pl
from jax.experimental.pallas import tpu as pltpu

T = 2048
D = 1024
E = 8
F = 2048
K = 2
BT = 256
NT = 24
NROWS = NT * BT
NF = 2
FB = F // NF


def _routing_kernel(logits_ref, scale_ref, p0_ref, p1_ref, w0_ref, w1_ref,
                    te_ref):
    logits = logits_ref[...]
    scale = scale_ref[...]
    iota_e = lax.broadcasted_iota(jnp.int32, (T, E), 1)

    m1 = jnp.max(logits, axis=1, keepdims=True)
    id1 = jnp.min(jnp.where(logits == m1, iota_e, E), axis=1, keepdims=True)
    sel1 = iota_e == id1
    masked = jnp.where(sel1, -jnp.inf, logits)
    m2 = jnp.max(masked, axis=1, keepdims=True)
    id2 = jnp.min(jnp.where(masked == m2, iota_e, E), axis=1, keepdims=True)
    sel2 = iota_e == id2

    ex = jnp.exp(logits - m1)
    probs = ex / jnp.sum(ex, axis=1, keepdims=True)
    gate = jnp.where(sel1 | sel2, probs, 0.0)
    renorm = jnp.sum(gate, axis=1, keepdims=True)
    renorm = jnp.where(renorm > 0.0, renorm, 1.0)
    ps = probs * scale
    w0 = jnp.sum(jnp.where(sel1, ps, 0.0), axis=1, keepdims=True) / renorm
    w1 = jnp.sum(jnp.where(sel2, ps, 0.0), axis=1, keepdims=True) / renorm

    oh1 = sel1.astype(jnp.float32)
    oh2 = sel2.astype(jnp.float32)

    def _cumsum0(a):
        s = 1
        while s < T:
            shifted = jnp.concatenate(
                [jnp.zeros((s, E), jnp.float32), a[: T - s, :]], axis=0)
            a = a + shifted
            s *= 2
        return a

    cs1 = _cumsum0(oh1)
    cs2 = _cumsum0(oh2)
    tot1 = cs1[T - 1:T, :]
    n_e = tot1 + cs2[T - 1:T, :]

    r0 = jnp.sum(jnp.where(sel1, cs1, 0.0), axis=1, keepdims=True) - 1.0
    r1 = (jnp.sum(jnp.where(sel2, cs2 + tot1, 0.0), axis=1, keepdims=True)
          - 1.0)

    nt_e = jnp.ceil(n_e / BT)
    strict_lt = (lax.broadcasted_iota(jnp.int32, (E, E), 0)
                 < lax.broadcasted_iota(jnp.int32, (E, E), 1))
    excl = jnp.dot(nt_e, strict_lt.astype(jnp.float32),
                   preferred_element_type=jnp.float32)
    p_start = excl * BT

    pos0 = jnp.sum(jnp.where(sel1, p_start, 0.0), axis=1, keepdims=True) + r0
    pos1 = jnp.sum(jnp.where(sel2, p_start, 0.0), axis=1, keepdims=True) + r1
    p0_ref[...] = pos0.astype(jnp.int32)
    p1_ref[...] = pos1.astype(jnp.int32)
    w0_ref[...] = w0
    w1_ref[...] = w1

    tile_iota = lax.broadcasted_iota(jnp.int32, (32, E), 0)
    cnt = jnp.sum((tile_iota >= excl.astype(jnp.int32)).astype(jnp.int32),
                  axis=1, keepdims=True) - 1
    te_ref[...] = jnp.clip(cnt, 0, E - 1)


def _routing(router_logits, per_expert_scale, interpret=False):
    out_shapes = (
        jax.ShapeDtypeStruct((T, 1), jnp.int32),
        jax.ShapeDtypeStruct((T, 1), jnp.int32),
        jax.ShapeDtypeStruct((T, 1), jnp.float32),
        jax.ShapeDtypeStruct((T, 1), jnp.float32),
        jax.ShapeDtypeStruct((32, 1), jnp.int32),
    )
    return pl.pallas_call(
        _routing_kernel,
        out_shape=out_shapes,
        interpret=interpret,
    )(router_logits, per_expert_scale.reshape(1, E))


def _gelu_exact(x):
    return 0.5 * x * (1.0 + lax.erf(x * 0.7071067811865476))


def _ffn1_kernel(te_ref, xs_ref, wg_ref, wu_ref, act_ref):
    x = xs_ref[...]
    g = jnp.dot(x, wg_ref[0], preferred_element_type=jnp.float32)
    u = jnp.dot(x, wu_ref[0], preferred_element_type=jnp.float32)
    act_ref[...] = _gelu_exact(g) * u


def _ffn2_kernel(te_ref, act_ref, wd_ref, sw_ref, y_ref):
    a = act_ref[...]
    y = jnp.dot(a, wd_ref[0], preferred_element_type=jnp.float32)
    y_ref[...] = y * sw_ref[...]


def _ffn(xs, w_gate, w_up, w_down, sorted_w, te, interpret=False):
    act = pl.pallas_call(
        _ffn1_kernel,
        grid_spec=pltpu.PrefetchScalarGridSpec(
            num_scalar_prefetch=1,
            grid=(NF, NT),
            in_specs=[
                pl.BlockSpec((BT, D), lambda f, t, te: (t, 0)),
                pl.BlockSpec((1, D, FB), lambda f, t, te: (te[t], 0, f)),
                pl.BlockSpec((1, D, FB), lambda f, t, te: (te[t], 0, f)),
            ],
            out_specs=pl.BlockSpec((BT, FB), lambda f, t, te: (t, f)),
        ),
        out_shape=jax.ShapeDtypeStruct((NROWS, F), jnp.float32),
        interpret=interpret,
    )(te, xs, w_gate, w_up)

    y = pl.pallas_call(
        _ffn2_kernel,
        grid_spec=pltpu.PrefetchScalarGridSpec(
            num_scalar_prefetch=1,
            grid=(NT,),
            in_specs=[
                pl.BlockSpec((BT, F), lambda t, te: (t, 0)),
                pl.BlockSpec((1, F, D), lambda t, te: (te[t], 0, 0)),
                pl.BlockSpec((BT, 1), lambda t, te: (t, 0)),
            ],
            out_specs=pl.BlockSpec((BT, D), lambda t, te: (t, 0)),
        ),
        out_shape=jax.ShapeDtypeStruct((NROWS, D), jnp.float32),
        interpret=interpret,
    )(te, act, w_down, sorted_w)
    return y


def kernel(x, router_logits, per_expert_scale, w_gate, w_up, w_down):
    p0, p1, w0, w1, te = _routing(router_logits, per_expert_scale)
    p0 = p0.reshape(T)
    p1 = p1.reshape(T)
    te = te.reshape(32)

    tok = jnp.arange(T, dtype=jnp.int32)
    pos = jnp.concatenate([p0, p1])
    sorted_tok = jnp.zeros((NROWS,), jnp.int32).at[pos].set(
        jnp.concatenate([tok, tok]))
    sorted_w = jnp.zeros((NROWS,), jnp.float32).at[pos].set(
        jnp.concatenate([w0.reshape(T), w1.reshape(T)]))
    xs = x[sorted_tok]

    y = _ffn(xs, w_gate, w_up, w_down, sorted_w.reshape(NROWS, 1), te)

    out = y[p0] + y[p1]
    return out
